# Initial kernel scaffold; baseline (speedup 1.0000x reference)
import jax
import jax.numpy as jnp
from jax.experimental import pallas as pl


def kernel(span_sentence_index, np_sentence_index, distance_embeddings):
    raise NotImplementedError("write your pallas kernel here")


# speedup vs baseline: 2.5506x; 2.5506x over previous
import functools

import jax
import jax.numpy as jnp
from jax import lax
from jax.experimental import pallas as pl
from jax.experimental.pallas import tpu as pltpu
from jax.experimental.pallas import tpu_sc as plsc

ROWS, SEQ = 16384, 50
CATEGORY, DIST_EMBED = 64, 32
B = ROWS * SEQ
NW = 32
BW = B // NW
C = 1024
NCHUNK = BW // C
L = 16

_mesh = plsc.VectorSubcoreMesh(core_axis_name="c", subcore_axis_name="s")


@functools.partial(
    pl.kernel,
    mesh=_mesh,
    compiler_params=pltpu.CompilerParams(needs_layout_passes=False),
    out_type=jax.ShapeDtypeStruct((B * DIST_EMBED,), jnp.float32),
    scratch_types=[
        pltpu.VMEM((CATEGORY * DIST_EMBED,), jnp.float32),
        pltpu.VMEM((C,), jnp.int32),
        pltpu.VMEM((C,), jnp.int32),
        pltpu.VMEM((C * DIST_EMBED,), jnp.float32),
    ],
)
def _lookup(span_hbm, np_hbm, table_hbm, out_hbm, table_v, a_v, b_v, rows_v):
    wid = lax.axis_index("s") * 2 + lax.axis_index("c")
    base = wid * BW
    pltpu.sync_copy(table_hbm, table_v)

    def chunk_body(ci, carry):
        off = base + ci * C
        pltpu.sync_copy(span_hbm.at[pl.ds(off, C)], a_v)
        pltpu.sync_copy(np_hbm.at[pl.ds(off, C)], b_v)

        def grp(j, c2):
            a = a_v[pl.ds(j * L, L)]
            b = b_v[pl.ds(j * L, L)]
            d = jnp.minimum(jnp.abs(a - b), CATEGORY - 1)
            g = d * DIST_EMBED
            s = (lax.iota(jnp.int32, L) + j * L) * DIST_EMBED
            for col in range(DIST_EMBED):
                vals = plsc.load_gather(table_v, [g + col])
                plsc.store_scatter(rows_v, [s + col], vals)
            return c2

        lax.fori_loop(0, C // L, grp, 0)
        pltpu.sync_copy(rows_v, out_hbm.at[pl.ds(off * DIST_EMBED, C * DIST_EMBED)])
        return carry

    lax.fori_loop(0, NCHUNK, chunk_body, 0)


def kernel(span_sentence_index, np_sentence_index, distance_embeddings):
    span = span_sentence_index.reshape(-1)
    npi = np_sentence_index.reshape(-1)
    out = _lookup(span, npi, distance_embeddings.reshape(-1))
    return out.reshape(ROWS, SEQ, DIST_EMBED)
